# baseline (speedup 1.0000x reference)
import functools

import jax
import jax.numpy as jnp
from jax import lax
from jax.experimental import pallas as pl
from jax.experimental.pallas import tpu as pltpu
from jax.experimental.pallas import tpu_sc as plsc

TRAJ = 16384
EMB = 64
NC = 2
NS = 16
NW = NC * NS
ROWS_PER_W = TRAJ // NW
CHUNK = 128
NCHUNK = ROWS_PER_W // CHUNK


def _sc_gather(idx2d, table):
    mesh = plsc.VectorSubcoreMesh(core_axis_name="c", subcore_axis_name="s")

    @functools.partial(
        pl.kernel,
        mesh=mesh,
        out_type=jax.ShapeDtypeStruct((TRAJ, EMB), jnp.float32),
        scratch_types=[
            pltpu.VMEM((NCHUNK, CHUNK), jnp.int32),
            pltpu.VMEM((ROWS_PER_W, EMB), jnp.float32),
            pltpu.SemaphoreType.DMA,
        ],
        compiler_params=pltpu.CompilerParams(use_tc_tiling_on_sc=False),
    )
    def k(idx_hbm, table_hbm, out_hbm, idx_v, rows_v, sem):
        wid = lax.axis_index("s") * NC + lax.axis_index("c")
        pltpu.sync_copy(idx_hbm.at[pl.ds(wid * NCHUNK, NCHUNK)], idx_v)
        copies = [
            pltpu.make_async_copy(
                table_hbm.at[idx_v.at[j]],
                rows_v.at[pl.ds(j * CHUNK, CHUNK)],
                sem,
            )
            for j in range(NCHUNK)
        ]
        for c in copies:
            c.start()
        for c in copies:
            c.wait()
        pltpu.sync_copy(rows_v, out_hbm.at[pl.ds(wid * ROWS_PER_W, ROWS_PER_W)])

    return k(idx2d, table)


def _tc_linear_lrelu(x, W, b2d):
    def body(x_ref, w_ref, b_ref, o_ref):
        y = lax.dot_general(
            x_ref[...], w_ref[...],
            (((1,), (1,)), ((), ())),
            preferred_element_type=jnp.float32,
        )
        y = y + b_ref[...]
        o_ref[...] = jnp.where(y >= 0, y, 0.01 * y)

    grid = 8
    rows = TRAJ // grid
    return pl.pallas_call(
        body,
        grid=(grid,),
        in_specs=[
            pl.BlockSpec((rows, EMB), lambda i: (i, 0)),
            pl.BlockSpec((EMB, EMB), lambda i: (0, 0)),
            pl.BlockSpec((1, EMB), lambda i: (0, 0)),
        ],
        out_specs=pl.BlockSpec((rows, EMB), lambda i: (i, 0)),
        out_shape=jax.ShapeDtypeStruct((TRAJ, EMB), jnp.float32),
    )(x, W, b2d)


def kernel(acs, emb_table, W, b):
    idx2d = jnp.reshape(acs.astype(jnp.int32), (TRAJ // CHUNK, CHUNK))
    x = _sc_gather(idx2d, emb_table)
    out = _tc_linear_lrelu(x, W, jnp.reshape(b, (1, EMB)))
    return jnp.reshape(out, (TRAJ, 1, EMB))


# speedup vs baseline: 1.4056x; 1.4056x over previous
import functools

import jax
import jax.numpy as jnp
from jax import lax
from jax.experimental import pallas as pl
from jax.experimental.pallas import tpu as pltpu
from jax.experimental.pallas import tpu_sc as plsc

TRAJ = 16384
EMB = 64
VOCAB = 100000
NC = 2
NS = 16
NW = NC * NS
ROWS_PER_W = TRAJ // NW
CHUNK = 128
NCHUNK = ROWS_PER_W // CHUNK

TBLK = 1024
TGRID = 49
PACKED_ROWS = TGRID * TBLK
PACKED_FLAT = 2 * PACKED_ROWS


def _lrelu(y):
    return jnp.where(y >= 0, y, 0.01 * y)


def _tc_transform(tableT, W, b2d):
    def body(ta_ref, tb_ref, w_ref, b_ref, o_ref):
        w = w_ref[...]
        bias = b_ref[...]
        ya = lax.dot_general(ta_ref[...], w, (((0,), (1,)), ((), ())),
                             preferred_element_type=jnp.float32)
        yb = lax.dot_general(tb_ref[...], w, (((0,), (1,)), ((), ())),
                             preferred_element_type=jnp.float32)
        o_ref[:, 0:EMB] = _lrelu(ya + bias)
        o_ref[:, EMB:2 * EMB] = _lrelu(yb + bias)

    return pl.pallas_call(
        body,
        grid=(TGRID,),
        in_specs=[
            pl.BlockSpec((EMB, TBLK), lambda i: (0, 2 * i)),
            pl.BlockSpec((EMB, TBLK), lambda i: (0, 2 * i + 1)),
            pl.BlockSpec((EMB, EMB), lambda i: (0, 0)),
            pl.BlockSpec((1, EMB), lambda i: (0, 0)),
        ],
        out_specs=pl.BlockSpec((TBLK, 2 * EMB), lambda i: (i, 0)),
        out_shape=jax.ShapeDtypeStruct((PACKED_ROWS, 2 * EMB), jnp.float32),
    )(tableT, tableT, W, b2d)


def _sc_gather(idx2d, packed_flat):
    mesh = plsc.VectorSubcoreMesh(core_axis_name="c", subcore_axis_name="s")

    @functools.partial(
        pl.kernel,
        mesh=mesh,
        out_type=jax.ShapeDtypeStruct((TRAJ, 2 * EMB), jnp.float32),
        scratch_types=[
            pltpu.VMEM((NCHUNK, CHUNK), jnp.int32),
            pltpu.VMEM((NCHUNK, CHUNK), jnp.int32),
            pltpu.VMEM((ROWS_PER_W, EMB), jnp.float32),
            pltpu.SemaphoreType.DMA,
        ],
        compiler_params=pltpu.CompilerParams(use_tc_tiling_on_sc=False),
    )
    def k(idx_hbm, table_hbm, out_hbm, idx_raw, idx_t, rows_v, sem):
        wid = lax.axis_index("s") * NC + lax.axis_index("c")
        base = wid * ROWS_PER_W
        pltpu.sync_copy(idx_hbm.at[pl.ds(wid * NCHUNK, NCHUNK)], idx_raw)
        for c in range(NCHUNK):
            for kk in range(CHUNK // 16):
                v = idx_raw[c, pl.ds(kk * 16, 16)]
                grp = jnp.bitwise_and(v, jnp.int32(~2047))
                loc = jnp.left_shift(jnp.bitwise_and(v, jnp.int32(1023)), 1)
                par = jnp.bitwise_and(jnp.right_shift(v, 10), jnp.int32(1))
                idx_t[c, pl.ds(kk * 16, 16)] = grp + loc + par
        copies = [
            pltpu.make_async_copy(
                table_hbm.at[idx_t.at[c]],
                rows_v.at[pl.ds(c * CHUNK, CHUNK)],
                sem,
            )
            for c in range(NCHUNK)
        ]
        for cp in copies:
            cp.start()
        for cp in copies:
            cp.wait()
        pltpu.sync_copy(
            rows_v, out_hbm.at[pl.ds(base, ROWS_PER_W), pl.ds(0, EMB)]
        )

    return k(idx2d, packed_flat)


def _tc_transpose(g):
    def body(g_ref, o_ref):
        o_ref[0] = g_ref[:, 0:EMB].T

    blk = 2048
    return pl.pallas_call(
        body,
        grid=(TRAJ // blk,),
        in_specs=[pl.BlockSpec((blk, 2 * EMB), lambda i: (i, 0))],
        out_specs=pl.BlockSpec((1, EMB, blk), lambda i: (0, 0, i)),
        out_shape=jax.ShapeDtypeStruct((1, EMB, TRAJ), jnp.float32),
    )(g)


def kernel(acs, emb_table, W, b):
    tableT = jnp.transpose(emb_table)
    packed = _tc_transform(tableT, W, jnp.reshape(b, (1, EMB)))
    packed_flat = jnp.reshape(packed, (PACKED_FLAT, EMB))
    idx2d = jnp.reshape(acs.astype(jnp.int32), (TRAJ // CHUNK, CHUNK))
    g = _sc_gather(idx2d, packed_flat)
    out = _tc_transpose(g)
    return jnp.transpose(out, (2, 0, 1))
